# baseline (speedup 1.0000x reference)
import jax
import jax.numpy as jnp
from jax import lax
from jax.experimental import pallas as pl
from jax.experimental.pallas import tpu as pltpu

_LEVELS = ((8, 80), (16, 40), (32, 20))
_NCH = 85
_TOTAL = sum(h * h for _, h in _LEVELS)


def _decode_body(in0a_ref, in0b_ref, in1_ref, in2_ref,
                 xs_ref, ys_ref, es_ref, out_ref):
    b = pl.program_id(0)
    eye = (lax.broadcasted_iota(jnp.int32, (_NCH, _NCH), 0)
           == lax.broadcasted_iota(jnp.int32, (_NCH, _NCH), 1)).astype(jnp.float32)
    off = 0
    v0 = jnp.concatenate([in0a_ref[0], in0b_ref[0]], axis=0)
    for v, (stride, hsize) in zip((v0, in1_ref[0], in2_ref[0]), _LEVELS):
        hw = hsize * hsize
        pos = lax.broadcasted_iota(jnp.int32, (1, hw), 1)
        gx = (pos % hsize).astype(jnp.float32)
        gy = (pos // hsize).astype(jnp.float32)
        row = lax.broadcasted_iota(jnp.int32, (_NCH, hw), 0)
        s = jnp.float32(stride)
        dec = jnp.where(
            row == 0, (v + gx) * s,
            jnp.where(row == 1, (v + gy) * s,
                      jnp.where(row < 4, jnp.exp(v) * s, v)))
        out_ref[0, pl.ds(off, hw), :] = dec.T

        @pl.when(b == 0)
        def _():
            xs_ref[0, pl.ds(off, hw)] = gx[0]
            ys_ref[0, pl.ds(off, hw)] = gy[0]
            es_ref[0, pl.ds(off, hw)] = jnp.full((hw,), s, jnp.float32)

        off += hw


def kernel(output0, output1, output2):
    batch = output0.shape[0]
    out_shapes = (
        jax.ShapeDtypeStruct((1, _TOTAL), jnp.float32),
        jax.ShapeDtypeStruct((1, _TOTAL), jnp.float32),
        jax.ShapeDtypeStruct((1, _TOTAL), jnp.float32),
        jax.ShapeDtypeStruct((batch, _TOTAL, _NCH), jnp.float32),
    )
    in_specs = [
        pl.BlockSpec((1, 48, 6400), lambda b: (b, 0, 0)),
        pl.BlockSpec((1, 37, 6400), lambda b: (b, 0, 0)),
        pl.BlockSpec((1, _NCH, 1600), lambda b: (b, 0, 0)),
        pl.BlockSpec((1, _NCH, 400), lambda b: (b, 0, 0)),
    ]
    out_specs = (
        pl.BlockSpec((1, _TOTAL), lambda b: (0, 0)),
        pl.BlockSpec((1, _TOTAL), lambda b: (0, 0)),
        pl.BlockSpec((1, _TOTAL), lambda b: (0, 0)),
        pl.BlockSpec((1, _TOTAL, _NCH), lambda b: (b, 0, 0)),
    )
    _call = pl.pallas_call(
        _decode_body,
        grid=(batch,),
        in_specs=in_specs,
        out_specs=out_specs,
        out_shape=out_shapes,
        compiler_params=pltpu.CompilerParams(
            dimension_semantics=("parallel",)),
    )
    r0 = output0.reshape(batch, _NCH, -1)
    xs, ys, es, outputs = _call(r0[:, :48], r0[:, 48:],
                                output1.reshape(batch, _NCH, -1),
                                output2.reshape(batch, _NCH, -1))
    return (xs, ys, es, outputs)


# speedup vs baseline: 1.3263x; 1.3263x over previous
import jax
import jax.numpy as jnp
from jax import lax
from jax.experimental import pallas as pl
from jax.experimental.pallas import tpu as pltpu

_NCH = 85
_TOTAL = 8400
_NCHUNK = 2
_CHUNK0 = 6400 // _NCHUNK


def _decode(v, hsize, stride, pos0):
    n = v.shape[1]
    pos = lax.broadcasted_iota(jnp.int32, (1, n), 1) + jnp.int32(pos0)
    gx = (pos % hsize).astype(jnp.float32)
    gy = (pos // hsize).astype(jnp.float32)
    row = lax.broadcasted_iota(jnp.int32, (_NCH, n), 0)
    s = jnp.float32(stride)
    return jnp.where(
        row == 0, (v + gx) * s,
        jnp.where(row == 1, (v + gy) * s,
                  jnp.where(row < 4, jnp.exp(v) * s, v)))


def _shift_arrays(xs_ref, ys_ref, es_ref, base, hw, hsize, stride):
    fpos = lax.broadcasted_iota(jnp.int32, (hw,), 0)
    xs_ref[0, pl.ds(base, hw)] = (fpos % hsize).astype(jnp.float32)
    ys_ref[0, pl.ds(base, hw)] = (fpos // hsize).astype(jnp.float32)
    es_ref[0, pl.ds(base, hw)] = jnp.full((hw,), stride, jnp.float32)


def _decode_body(in0_ref, in1_ref, in2_ref,
                 xs_ref, ys_ref, es_ref, out_ref):
    b = pl.program_id(0)
    j = pl.program_id(1)

    dec0 = _decode(in0_ref[0], 80, 8, j * jnp.int32(_CHUNK0))
    out_ref[0, pl.ds(j * _CHUNK0, _CHUNK0), :] = dec0.T

    @pl.when(j == 0)
    def _():
        dec1 = _decode(in1_ref[0], 40, 16, 0)
        out_ref[0, pl.ds(6400, 1600), :] = dec1.T
        dec2 = _decode(in2_ref[0], 20, 32, 0)
        out_ref[0, pl.ds(8000, 400), :] = dec2.T

    @pl.when(jnp.logical_and(b == 0, j == 0))
    def _():
        _shift_arrays(xs_ref, ys_ref, es_ref, 0, 6400, 80, 8)
        _shift_arrays(xs_ref, ys_ref, es_ref, 6400, 1600, 40, 16)
        _shift_arrays(xs_ref, ys_ref, es_ref, 8000, 400, 20, 32)


def kernel(output0, output1, output2):
    batch = output0.shape[0]
    out_shapes = (
        jax.ShapeDtypeStruct((1, _TOTAL), jnp.float32),
        jax.ShapeDtypeStruct((1, _TOTAL), jnp.float32),
        jax.ShapeDtypeStruct((1, _TOTAL), jnp.float32),
        jax.ShapeDtypeStruct((batch, _TOTAL, _NCH), jnp.float32),
    )
    in_specs = [
        pl.BlockSpec((1, _NCH, _CHUNK0), lambda b, j: (b, 0, j)),
        pl.BlockSpec((1, _NCH, 1600), lambda b, j: (b, 0, 0)),
        pl.BlockSpec((1, _NCH, 400), lambda b, j: (b, 0, 0)),
    ]
    out_specs = (
        pl.BlockSpec((1, _TOTAL), lambda b, j: (0, 0)),
        pl.BlockSpec((1, _TOTAL), lambda b, j: (0, 0)),
        pl.BlockSpec((1, _TOTAL), lambda b, j: (0, 0)),
        pl.BlockSpec((1, _TOTAL, _NCH), lambda b, j: (b, 0, 0)),
    )
    xs, ys, es, outputs = pl.pallas_call(
        _decode_body,
        grid=(batch, _NCHUNK),
        in_specs=in_specs,
        out_specs=out_specs,
        out_shape=out_shapes,
        compiler_params=pltpu.CompilerParams(
            dimension_semantics=("parallel", "arbitrary")),
    )(output0.reshape(batch, _NCH, -1),
      output1.reshape(batch, _NCH, -1),
      output2.reshape(batch, _NCH, -1))
    return (xs, ys, es, outputs)


# speedup vs baseline: 1.3825x; 1.0423x over previous
import jax
import jax.numpy as jnp
from jax import lax
from jax.experimental import pallas as pl
from jax.experimental.pallas import tpu as pltpu

_LEVELS = ((8, 80), (16, 40), (32, 20))
_NCH = 85
_TOTAL = sum(h * h for _, h in _LEVELS)


def _decode_body(in0_ref, in1_ref, in2_ref,
                 xs_ref, ys_ref, es_ref, out_ref):
    b = pl.program_id(0)
    off = 0
    for in_ref, (stride, hsize) in zip((in0_ref, in1_ref, in2_ref), _LEVELS):
        hw = hsize * hsize
        v = in_ref[0]
        pos = lax.broadcasted_iota(jnp.int32, (1, hw), 1)
        gx = (pos % hsize).astype(jnp.float32)
        gy = (pos // hsize).astype(jnp.float32)
        row = lax.broadcasted_iota(jnp.int32, (_NCH, hw), 0)
        s = jnp.float32(stride)
        dec = jnp.where(
            row == 0, (v + gx) * s,
            jnp.where(row == 1, (v + gy) * s,
                      jnp.where(row < 4, jnp.exp(v) * s, v)))
        out_ref[0, pl.ds(off, hw), :] = dec.T

        @pl.when(b == 0)
        def _():
            xs_ref[0, pl.ds(off, hw)] = gx[0]
            ys_ref[0, pl.ds(off, hw)] = gy[0]
            es_ref[0, pl.ds(off, hw)] = jnp.full((hw,), s, jnp.float32)

        off += hw


def kernel(output0, output1, output2):
    batch = output0.shape[0]
    out_shapes = (
        jax.ShapeDtypeStruct((1, _TOTAL), jnp.float32),
        jax.ShapeDtypeStruct((1, _TOTAL), jnp.float32),
        jax.ShapeDtypeStruct((1, _TOTAL), jnp.float32),
        jax.ShapeDtypeStruct((batch, _TOTAL, _NCH), jnp.float32),
    )
    in_specs = [
        pl.BlockSpec((1, _NCH, h * h), lambda b: (b, 0, 0))
        for _, h in _LEVELS
    ]
    out_specs = (
        pl.BlockSpec((1, _TOTAL), lambda b: (0, 0)),
        pl.BlockSpec((1, _TOTAL), lambda b: (0, 0)),
        pl.BlockSpec((1, _TOTAL), lambda b: (0, 0)),
        pl.BlockSpec((1, _TOTAL, _NCH), lambda b: (b, 0, 0)),
    )
    xs, ys, es, outputs = pl.pallas_call(
        _decode_body,
        grid=(batch,),
        in_specs=in_specs,
        out_specs=out_specs,
        out_shape=out_shapes,
        compiler_params=pltpu.CompilerParams(
            dimension_semantics=("parallel",)),
    )(output0.reshape(batch, _NCH, -1),
      output1.reshape(batch, _NCH, -1),
      output2.reshape(batch, _NCH, -1))
    return (xs, ys, es, outputs)
